# initial kernel scaffold (unmeasured)
import numpy as np
import jax
import jax.numpy as jnp
from jax import lax
from jax.experimental import pallas as pl
from jax.experimental.pallas import tpu as pltpu

N_DEV = 4
SQ = 1024
DM = 1024
GW = 1024
HG = 8
DH = 128
NR = 4
RB = SQ // NR
SCALE = 0.08838834764831843

_ar = np.arange(SQ)
_SRC = (4 * ((_ar % 256) // 64) + _ar // 256) * 64 + _ar % 64
_INV = np.zeros(SQ, dtype=np.int32)
_INV[_SRC] = _ar


def _attn_group(qbuf, k_ref, v_ref, ctx, p):
    col0 = p * GW
    for h in range(HG):
        qc = h * DH
        kc = col0 + h * DH

        def r_body(r, _, qc=qc, kc=kc):
            row = r * RB
            q = qbuf[pl.ds(row, RB), qc:qc + DH]
            k = k_ref[pl.ds(row, RB), kc:kc + DH]
            s = lax.dot_general(
                q, k, (((1,), (1,)), ((), ())),
                preferred_element_type=jnp.float32,
            )
            m = jnp.max(s, axis=1, keepdims=True)
            e = jnp.exp(s - m)
            w = (e / jnp.sum(e, axis=1, keepdims=True)).astype(jnp.bfloat16)
            v = v_ref[pl.ds(row, RB), kc:kc + DH]
            ctx[pl.ds(row, RB), kc:kc + DH] = jnp.dot(
                w, v, preferred_element_type=jnp.float32
            ).astype(jnp.bfloat16)
            return 0

        lax.fori_loop(0, NR, r_body, 0)


def kernel(x, Wq, K_ext, V_ext, Wo):
    my = lax.axis_index("i")

    xb = (x[0] * SCALE).astype(jnp.bfloat16)[_SRC, :]
    wq = Wq.astype(jnp.bfloat16)
    wo = Wo.astype(jnp.bfloat16)

    blk = (my - jnp.arange(N_DEV)) % N_DEV
    k2 = lax.dynamic_index_in_dim(K_ext, my, 0, keepdims=False).reshape(SQ, N_DEV * GW)
    v2 = lax.dynamic_index_in_dim(V_ext, my, 0, keepdims=False).reshape(SQ, N_DEV * GW)
    kp = jnp.take(k2.reshape(SQ, N_DEV, GW), blk, axis=1).reshape(SQ, N_DEV * GW)
    vp = jnp.take(v2.reshape(SQ, N_DEV, GW), blk, axis=1).reshape(SQ, N_DEV * GW)
    kp = kp[_SRC, :].astype(jnp.bfloat16)
    vp = vp[_SRC, :].astype(jnp.bfloat16)

    def body(x_ref, wq_ref, wo_ref, k_ref, v_ref, out_ref,
             comm_wq, comm_wo, qbuf, ctx,
             sq_send, sq_recv, so_send, so_recv):
        me = lax.axis_index("i")
        right = lax.rem(me + 1, N_DEV)
        left = lax.rem(me + 3, N_DEV)

        barrier_sem = pltpu.get_barrier_semaphore()
        for nbr in (left, right):
            pl.semaphore_signal(
                barrier_sem, inc=1,
                device_id=(nbr,), device_id_type=pl.DeviceIdType.MESH,
            )
        pl.semaphore_wait(barrier_sem, 2)

        comm_wq[0, :, :] = wq_ref[:, :]
        comm_wo[0, :, :] = wo_ref[:, :]

        for h in (1, 2, 3):
            dq = pltpu.make_async_remote_copy(
                src_ref=comm_wq.at[h - 1],
                dst_ref=comm_wq.at[h],
                send_sem=sq_send.at[h - 1],
                recv_sem=sq_recv.at[h - 1],
                device_id=(right,),
                device_id_type=pl.DeviceIdType.MESH,
            )
            do = pltpu.make_async_remote_copy(
                src_ref=comm_wo.at[h - 1],
                dst_ref=comm_wo.at[h],
                send_sem=so_send.at[h - 1],
                recv_sem=so_recv.at[h - 1],
                device_id=(left,),
                device_id_type=pl.DeviceIdType.MESH,
            )
            dq.start()
            do.start()
            dq.wait()
            do.wait()

        for p in range(N_DEV):
            qbuf[:, :] = jnp.dot(
                x_ref[:, :], comm_wq[p, :, :],
                preferred_element_type=jnp.float32,
            ).astype(jnp.bfloat16)
            _attn_group(qbuf, k_ref, v_ref, ctx, p)

        for q in range(N_DEV):
            pblk = (N_DEV - q) % N_DEV
            contrib = jnp.dot(
                ctx[:, pblk * GW:(pblk + 1) * GW], comm_wo[q, :, :],
                preferred_element_type=jnp.float32,
            )
            if q == 0:
                out_ref[:, :] = contrib
            else:
                out_ref[:, :] = out_ref[:, :] + contrib

    out_p = pl.pallas_call(
        body,
        out_shape=jax.ShapeDtypeStruct((SQ, DM), jnp.float32),
        in_specs=[pl.BlockSpec(memory_space=pltpu.VMEM)] * 5,
        out_specs=pl.BlockSpec(memory_space=pltpu.VMEM),
        scratch_shapes=[
            pltpu.VMEM((N_DEV, DM, GW), jnp.bfloat16),
            pltpu.VMEM((N_DEV, GW, DM), jnp.bfloat16),
            pltpu.VMEM((SQ, GW), jnp.bfloat16),
            pltpu.VMEM((SQ, N_DEV * GW), jnp.bfloat16),
            pltpu.SemaphoreType.DMA((3,)),
            pltpu.SemaphoreType.DMA((3,)),
            pltpu.SemaphoreType.DMA((3,)),
            pltpu.SemaphoreType.DMA((3,)),
        ],
        compiler_params=pltpu.CompilerParams(collective_id=0),
    )(xb, wq, wo, kp, vp)

    return out_p[_INV, :][None]


# baseline (device time: 311118 ns/iter reference)
import numpy as np
import jax
import jax.numpy as jnp
from jax import lax
from jax.experimental import pallas as pl
from jax.experimental.pallas import tpu as pltpu

N_DEV = 4
SQ = 1024
DM = 1024
GW = 1024
HG = 8
DH = 128
NR = 4
RB = SQ // NR
SCALE = 0.08838834764831843

_ar = np.arange(SQ)
_SRC = (4 * ((_ar % 256) // 64) + _ar // 256) * 64 + _ar % 64
_INV = np.zeros(SQ, dtype=np.int32)
_INV[_SRC] = _ar


def _attn_group(qbuf, k_ref, v_ref, ctx, p):
    col0 = p * GW
    for h in range(HG):
        qc = h * DH
        kc = col0 + h * DH

        def r_body(r, _, qc=qc, kc=kc):
            row = r * RB
            q = qbuf[pl.ds(row, RB), qc:qc + DH]
            k = k_ref[pl.ds(row, RB), kc:kc + DH]
            s = lax.dot_general(
                q, k, (((1,), (1,)), ((), ())),
                preferred_element_type=jnp.float32,
            )
            m = jnp.max(s, axis=1, keepdims=True)
            e = jnp.exp(s - m)
            w = (e / jnp.sum(e, axis=1, keepdims=True)).astype(jnp.bfloat16)
            v = v_ref[pl.ds(row, RB), kc:kc + DH]
            ctx[pl.ds(row, RB), kc:kc + DH] = jnp.dot(
                w, v, preferred_element_type=jnp.float32
            ).astype(jnp.bfloat16)
            return 0

        lax.fori_loop(0, NR, r_body, 0)


def kernel(x, Wq, K_ext, V_ext, Wo):
    my = lax.axis_index("i")

    xb = (x[0] * SCALE).astype(jnp.bfloat16)[_SRC, :]
    wq = Wq.astype(jnp.bfloat16)
    wo = Wo.astype(jnp.bfloat16)

    blk = (my - jnp.arange(N_DEV)) % N_DEV
    k2 = lax.dynamic_index_in_dim(K_ext, my, 0, keepdims=False).reshape(SQ, N_DEV * GW)
    v2 = lax.dynamic_index_in_dim(V_ext, my, 0, keepdims=False).reshape(SQ, N_DEV * GW)
    kp = jnp.take(k2.reshape(SQ, N_DEV, GW), blk, axis=1).reshape(SQ, N_DEV * GW)
    vp = jnp.take(v2.reshape(SQ, N_DEV, GW), blk, axis=1).reshape(SQ, N_DEV * GW)
    kp = kp[_SRC, :].astype(jnp.bfloat16)
    vp = vp[_SRC, :].astype(jnp.bfloat16)

    def body(x_ref, wq_ref, wo_ref, k_ref, v_ref, out_ref,
             comm_wq, comm_wo, qbuf, ctx,
             sq_send, sq_recv, so_send, so_recv):
        me = lax.axis_index("i")
        right = lax.rem(me + 1, N_DEV)
        left = lax.rem(me + 3, N_DEV)

        barrier_sem = pltpu.get_barrier_semaphore()
        for nbr in (left, right):
            pl.semaphore_signal(
                barrier_sem, inc=1,
                device_id=(nbr,), device_id_type=pl.DeviceIdType.MESH,
            )
        pl.semaphore_wait(barrier_sem, 2)

        comm_wq[0, :, :] = wq_ref[:, :]
        comm_wo[0, :, :] = wo_ref[:, :]

        for h in (1, 2, 3):
            dq = pltpu.make_async_remote_copy(
                src_ref=comm_wq.at[h - 1],
                dst_ref=comm_wq.at[h],
                send_sem=sq_send.at[h - 1],
                recv_sem=sq_recv.at[h - 1],
                device_id=(right,),
                device_id_type=pl.DeviceIdType.MESH,
            )
            do = pltpu.make_async_remote_copy(
                src_ref=comm_wo.at[h - 1],
                dst_ref=comm_wo.at[h],
                send_sem=so_send.at[h - 1],
                recv_sem=so_recv.at[h - 1],
                device_id=(left,),
                device_id_type=pl.DeviceIdType.MESH,
            )
            dq.start()
            do.start()
            dq.wait()
            do.wait()

        for p in range(N_DEV):
            qbuf[:, :] = jnp.dot(
                x_ref[:, :], comm_wq[p, :, :],
                preferred_element_type=jnp.float32,
            ).astype(jnp.bfloat16)
            _attn_group(qbuf, k_ref, v_ref, ctx, p)

        for q in range(N_DEV):
            pblk = (N_DEV - q) % N_DEV
            contrib = jnp.dot(
                ctx[:, pblk * GW:(pblk + 1) * GW], comm_wo[q, :, :],
                preferred_element_type=jnp.float32,
            )
            if q == 0:
                out_ref[:, :] = contrib
            else:
                out_ref[:, :] = out_ref[:, :] + contrib

    out_p = pl.pallas_call(
        body,
        out_shape=jax.ShapeDtypeStruct((SQ, DM), jnp.float32),
        in_specs=[pl.BlockSpec(memory_space=pltpu.VMEM)] * 5,
        out_specs=pl.BlockSpec(memory_space=pltpu.VMEM),
        scratch_shapes=[
            pltpu.VMEM((N_DEV, DM, GW), jnp.bfloat16),
            pltpu.VMEM((N_DEV, GW, DM), jnp.bfloat16),
            pltpu.VMEM((SQ, GW), jnp.bfloat16),
            pltpu.VMEM((SQ, N_DEV * GW), jnp.bfloat16),
            pltpu.SemaphoreType.DMA((3,)),
            pltpu.SemaphoreType.DMA((3,)),
            pltpu.SemaphoreType.DMA((3,)),
            pltpu.SemaphoreType.DMA((3,)),
        ],
        compiler_params=pltpu.CompilerParams(
            collective_id=0,
            vmem_limit_bytes=60 * 1024 * 1024,
        ),
    )(xb, wq, wo, kp, vp)

    return out_p[_INV, :][None]


# device time: 281918 ns/iter; 1.1036x vs baseline; 1.1036x over previous
import numpy as np
import jax
import jax.numpy as jnp
from jax import lax
from jax.experimental import pallas as pl
from jax.experimental.pallas import tpu as pltpu

N_DEV = 4
SQ = 1024
DM = 1024
GW = 1024
HG = 8
DH = 128
NR = 4
RB = SQ // NR
SCALE = 0.08838834764831843

def _perm_rows(a):
    c = a.shape[1]
    return a.reshape(4, 4, 64, c).swapaxes(0, 1).reshape(SQ, c)


def _attn_group(qbuf, k_ref, v_ref, ctx, p):
    col0 = p * GW
    for h in range(HG):
        qc = h * DH
        kc = col0 + h * DH

        for r in range(NR):
            row = r * RB
            q = qbuf[row:row + RB, qc:qc + DH]
            k = k_ref[row:row + RB, kc:kc + DH]
            s = lax.dot_general(
                q, k, (((1,), (1,)), ((), ())),
                preferred_element_type=jnp.float32,
            )
            m = jnp.max(s, axis=1, keepdims=True)
            e = jnp.exp(s - m)
            w = (e / jnp.sum(e, axis=1, keepdims=True)).astype(jnp.bfloat16)
            v = v_ref[row:row + RB, kc:kc + DH]
            ctx[row:row + RB, kc:kc + DH] = jnp.dot(
                w, v, preferred_element_type=jnp.float32
            ).astype(jnp.bfloat16)


def kernel(x, Wq, K_ext, V_ext, Wo):
    my = lax.axis_index("i")

    xb = _perm_rows((x[0] * SCALE).astype(jnp.bfloat16))
    wq = Wq.astype(jnp.bfloat16)
    wo = Wo.astype(jnp.bfloat16)

    def _prep_kv(a):
        a3 = lax.dynamic_index_in_dim(a, my, 0, keepdims=False)
        a3 = a3.reshape(SQ, N_DEV, GW).astype(jnp.bfloat16)
        a3 = jnp.roll(a3[:, ::-1, :], my + 1, axis=1)
        return _perm_rows(a3.reshape(SQ, N_DEV * GW))

    kp = _prep_kv(K_ext)
    vp = _prep_kv(V_ext)

    def body(x_ref, wq_ref, wo_ref, k_ref, v_ref, out_ref,
             comm_wq, comm_wo, qbuf, ctx,
             sq_send, sq_recv, so_send, so_recv):
        me = lax.axis_index("i")
        right = lax.rem(me + 1, N_DEV)
        left = lax.rem(me + 3, N_DEV)

        barrier_sem = pltpu.get_barrier_semaphore()
        for nbr in (left, right):
            pl.semaphore_signal(
                barrier_sem, inc=1,
                device_id=(nbr,), device_id_type=pl.DeviceIdType.MESH,
            )
        pl.semaphore_wait(barrier_sem, 2)

        comm_wq[0, :, :] = wq_ref[:, :]
        comm_wo[0, :, :] = wo_ref[:, :]

        for h in (1, 2, 3):
            dq = pltpu.make_async_remote_copy(
                src_ref=comm_wq.at[h - 1],
                dst_ref=comm_wq.at[h],
                send_sem=sq_send.at[h - 1],
                recv_sem=sq_recv.at[h - 1],
                device_id=(right,),
                device_id_type=pl.DeviceIdType.MESH,
            )
            do = pltpu.make_async_remote_copy(
                src_ref=comm_wo.at[h - 1],
                dst_ref=comm_wo.at[h],
                send_sem=so_send.at[h - 1],
                recv_sem=so_recv.at[h - 1],
                device_id=(left,),
                device_id_type=pl.DeviceIdType.MESH,
            )
            dq.start()
            do.start()
            dq.wait()
            do.wait()

        for p in range(N_DEV):
            qbuf[:, :] = jnp.dot(
                x_ref[:, :], comm_wq[p, :, :],
                preferred_element_type=jnp.float32,
            ).astype(jnp.bfloat16)
            _attn_group(qbuf, k_ref, v_ref, ctx, p)

        for q in range(N_DEV):
            pblk = (N_DEV - q) % N_DEV
            contrib = jnp.dot(
                ctx[:, pblk * GW:(pblk + 1) * GW], comm_wo[q, :, :],
                preferred_element_type=jnp.float32,
            )
            if q == 0:
                out_ref[:, :] = contrib
            else:
                out_ref[:, :] = out_ref[:, :] + contrib

    out_p = pl.pallas_call(
        body,
        out_shape=jax.ShapeDtypeStruct((SQ, DM), jnp.float32),
        in_specs=[pl.BlockSpec(memory_space=pltpu.VMEM)] * 5,
        out_specs=pl.BlockSpec(memory_space=pltpu.VMEM),
        scratch_shapes=[
            pltpu.VMEM((N_DEV, DM, GW), jnp.bfloat16),
            pltpu.VMEM((N_DEV, GW, DM), jnp.bfloat16),
            pltpu.VMEM((SQ, GW), jnp.bfloat16),
            pltpu.VMEM((SQ, N_DEV * GW), jnp.bfloat16),
            pltpu.SemaphoreType.DMA((3,)),
            pltpu.SemaphoreType.DMA((3,)),
            pltpu.SemaphoreType.DMA((3,)),
            pltpu.SemaphoreType.DMA((3,)),
        ],
        compiler_params=pltpu.CompilerParams(
            collective_id=0,
            vmem_limit_bytes=60 * 1024 * 1024,
        ),
    )(xb, wq, wo, kp, vp)

    return _perm_rows(out_p)[None]


# device time: 241755 ns/iter; 1.2869x vs baseline; 1.1661x over previous
import jax
import jax.numpy as jnp
from jax import lax
from jax.experimental import pallas as pl
from jax.experimental.pallas import tpu as pltpu

N_DEV = 4
SQ = 1024
DM = 1024
GW = 1024
HG = 8
DH = 128
NR = 4
RB = SQ // NR
NB = 16
SCALE = 0.08838834764831843


def _perm_rows(a):
    c = a.shape[1]
    return a.reshape(4, 4, 64, c).swapaxes(0, 1).reshape(SQ, c)


def _attn_group(qbuf, kp, vp, ctx, p):
    col0 = p * GW
    for h in range(HG):
        qc = h * DH
        kc = col0 + h * DH
        for r in range(NR):
            row = r * RB
            q = qbuf[row:row + RB, qc:qc + DH]
            k = kp[row:row + RB, kc:kc + DH]
            s = lax.dot_general(
                q, k, (((1,), (1,)), ((), ())),
                preferred_element_type=jnp.float32,
            )
            m = jnp.max(s, axis=1, keepdims=True)
            e = jnp.exp(s - m)
            w = (e / jnp.sum(e, axis=1, keepdims=True)).astype(jnp.bfloat16)
            v = vp[row:row + RB, kc:kc + DH]
            ctx[row:row + RB, kc:kc + DH] = jnp.dot(
                w, v, preferred_element_type=jnp.float32
            ).astype(jnp.bfloat16)


def _shuffle_block(stage, slot, dst, p):
    c0 = p * GW
    for b in range(NB):
        qi, r = b // 4, b % 4
        d0 = r * RB + qi * 64
        dst[d0:d0 + 64, c0:c0 + GW] = (
            stage[slot, b * 64:(b + 1) * 64, :].astype(jnp.bfloat16))


def kernel(x, Wq, K_ext, V_ext, Wo):
    xb = _perm_rows((x[0] * SCALE).astype(jnp.bfloat16))
    wq = Wq.astype(jnp.bfloat16)
    wo = Wo.astype(jnp.bfloat16)
    k2 = K_ext.reshape(N_DEV, SQ, N_DEV * GW)
    v2 = V_ext.reshape(N_DEV, SQ, N_DEV * GW)

    def body(x_ref, wq_ref, wo_ref, k_hbm, v_hbm, out_ref,
             comm_wq, comm_wo, kp, vp, stage, qbuf, ctx,
             sq_send, sq_recv, so_send, so_recv, kv_sems):
        me = lax.axis_index("i")
        right = lax.rem(me + 1, N_DEV)
        left = lax.rem(me + 3, N_DEV)

        barrier_sem = pltpu.get_barrier_semaphore()
        for nbr in (left, right):
            pl.semaphore_signal(
                barrier_sem, inc=1,
                device_id=(nbr,), device_id_type=pl.DeviceIdType.MESH,
            )
        pl.semaphore_wait(barrier_sem, 2)

        def hop(h):
            dq = pltpu.make_async_remote_copy(
                src_ref=wq_ref if h == 1 else comm_wq.at[h - 2],
                dst_ref=comm_wq.at[h - 1],
                send_sem=sq_send.at[h - 1],
                recv_sem=sq_recv.at[h - 1],
                device_id=(right,),
                device_id_type=pl.DeviceIdType.MESH,
            )
            do = pltpu.make_async_remote_copy(
                src_ref=wo_ref if h == 1 else comm_wo.at[h - 2],
                dst_ref=comm_wo.at[h - 1],
                send_sem=so_send.at[h - 1],
                recv_sem=so_recv.at[h - 1],
                device_id=(left,),
                device_id_type=pl.DeviceIdType.MESH,
            )
            dq.start()
            do.start()
            return dq, do

        def kv_dma(src, p, slot):
            g = lax.rem(me - p + N_DEV, N_DEV)
            d = pltpu.make_async_copy(
                src.at[me, :, pl.ds(g * GW, GW)],
                stage.at[slot],
                kv_sems.at[slot],
            )
            d.start()
            return d

        def wq_at(p):
            return wq_ref[:, :] if p == 0 else comm_wq[p - 1, :, :]

        def wo_at(q):
            return wo_ref[:, :] if q == 0 else comm_wo[q - 1, :, :]

        def attn(p):
            qbuf[:, :] = jnp.dot(
                x_ref[:, :], wq_at(p), preferred_element_type=jnp.float32
            ).astype(jnp.bfloat16)
            _attn_group(qbuf, kp, vp, ctx, p)

        def outproj(q, first=False):
            pblk = (N_DEV - q) % N_DEV
            contrib = jnp.dot(
                ctx[:, pblk * GW:(pblk + 1) * GW], wo_at(q),
                preferred_element_type=jnp.float32,
            )
            if first:
                out_ref[:, :] = contrib
            else:
                out_ref[:, :] = out_ref[:, :] + contrib

        dq1, do1 = hop(1)
        dk0 = kv_dma(k_hbm, 0, 0)
        dv0 = kv_dma(v_hbm, 0, 1)
        dk0.wait()
        _shuffle_block(stage, 0, kp, 0)
        dk1 = kv_dma(k_hbm, 1, 0)
        dv0.wait()
        _shuffle_block(stage, 1, vp, 0)
        dv1 = kv_dma(v_hbm, 1, 1)

        attn(0)
        outproj(0, first=True)

        dq1.wait_recv()
        do1.wait_recv()
        dq2, do2 = hop(2)
        dk1.wait()
        _shuffle_block(stage, 0, kp, 1)
        dk2 = kv_dma(k_hbm, 2, 0)
        dv1.wait()
        _shuffle_block(stage, 1, vp, 1)
        dv2 = kv_dma(v_hbm, 2, 1)

        attn(1)

        dq2.wait_recv()
        do2.wait_recv()
        dq3, do3 = hop(3)
        dk2.wait()
        _shuffle_block(stage, 0, kp, 2)
        dk3 = kv_dma(k_hbm, 3, 0)
        dv2.wait()
        _shuffle_block(stage, 1, vp, 2)
        dv3 = kv_dma(v_hbm, 3, 1)

        attn(2)
        outproj(2)

        dq3.wait_recv()
        do3.wait_recv()
        dk3.wait()
        _shuffle_block(stage, 0, kp, 3)
        dv3.wait()
        _shuffle_block(stage, 1, vp, 3)

        attn(3)
        outproj(1)
        outproj(3)

        for d in (dq1, do1, dq2, do2, dq3, do3):
            d.wait_send()

    out_p = pl.pallas_call(
        body,
        out_shape=jax.ShapeDtypeStruct((SQ, DM), jnp.float32),
        in_specs=[
            pl.BlockSpec(memory_space=pltpu.VMEM),
            pl.BlockSpec(memory_space=pltpu.VMEM),
            pl.BlockSpec(memory_space=pltpu.VMEM),
            pl.BlockSpec(memory_space=pltpu.MemorySpace.HBM),
            pl.BlockSpec(memory_space=pltpu.MemorySpace.HBM),
        ],
        out_specs=pl.BlockSpec(memory_space=pltpu.VMEM),
        scratch_shapes=[
            pltpu.VMEM((3, DM, GW), jnp.bfloat16),
            pltpu.VMEM((3, GW, DM), jnp.bfloat16),
            pltpu.VMEM((SQ, N_DEV * GW), jnp.bfloat16),
            pltpu.VMEM((SQ, N_DEV * GW), jnp.bfloat16),
            pltpu.VMEM((2, SQ, GW), jnp.float32),
            pltpu.VMEM((SQ, GW), jnp.bfloat16),
            pltpu.VMEM((SQ, N_DEV * GW), jnp.bfloat16),
            pltpu.SemaphoreType.DMA((3,)),
            pltpu.SemaphoreType.DMA((3,)),
            pltpu.SemaphoreType.DMA((3,)),
            pltpu.SemaphoreType.DMA((3,)),
            pltpu.SemaphoreType.DMA((2,)),
        ],
        compiler_params=pltpu.CompilerParams(
            collective_id=0,
            vmem_limit_bytes=62 * 1024 * 1024,
        ),
    )(xb, wq, wo, k2, v2)

    return _perm_rows(out_p)[None]


# device time: 237997 ns/iter; 1.3072x vs baseline; 1.0158x over previous
import jax
import jax.numpy as jnp
from jax import lax
from jax.experimental import pallas as pl
from jax.experimental.pallas import tpu as pltpu

import os
_ABLATE_ATTN = os.environ.get("ABLATE_ATTN") == "1"

N_DEV = 4
SQ = 1024
DM = 1024
GW = 1024
HG = 8
DH = 128
NR = 4
RB = SQ // NR
NB = 16
SCALE = 0.08838834764831843


def _perm_rows(a):
    c = a.shape[1]
    return a.reshape(4, 4, 64, c).swapaxes(0, 1).reshape(SQ, c)


def _attn_group(qbuf, kp, vp, ctx, p):
    col0 = p * GW
    for h in range(HG):
        qc = h * DH
        kc = col0 + h * DH
        for r in range(NR):
            row = r * RB
            if _ABLATE_ATTN:
                ctx[row:row + RB, kc:kc + DH] = (
                    qbuf[row:row + RB, qc:qc + DH]
                    + kp[row:row + RB, kc:kc + DH]
                    + vp[row:row + RB, kc:kc + DH])
                continue
            q = qbuf[row:row + RB, qc:qc + DH]
            k = kp[row:row + RB, kc:kc + DH]
            s = lax.dot_general(
                q, k, (((1,), (1,)), ((), ())),
                preferred_element_type=jnp.float32,
            )
            m = jnp.max(s, axis=1, keepdims=True)
            e = jnp.exp(s - m)
            w = (e / jnp.sum(e, axis=1, keepdims=True)).astype(jnp.bfloat16)
            v = vp[row:row + RB, kc:kc + DH]
            ctx[row:row + RB, kc:kc + DH] = jnp.dot(
                w, v, preferred_element_type=jnp.float32
            ).astype(jnp.bfloat16)


def _shuffle_block(stage, slot, dst, p):
    c0 = p * GW
    for b in range(NB):
        qi, r = b // 4, b % 4
        d0 = r * RB + qi * 64
        dst[d0:d0 + 64, c0:c0 + GW] = (
            stage[slot, b * 64:(b + 1) * 64, :].astype(jnp.bfloat16))


def kernel(x, Wq, K_ext, V_ext, Wo):
    xb = _perm_rows((x[0] * SCALE).astype(jnp.bfloat16))
    wq = Wq.astype(jnp.bfloat16)
    wo = Wo.astype(jnp.bfloat16)
    k2 = K_ext.reshape(N_DEV, SQ, N_DEV * GW)
    v2 = V_ext.reshape(N_DEV, SQ, N_DEV * GW)

    def body(x_ref, wq_ref, wo_ref, k_hbm, v_hbm, out_ref,
             comm_wq, comm_wo, kp, vp, stage, qbuf, ctx,
             sq_send, sq_recv, so_send, so_recv, kv_sems):
        me = lax.axis_index("i")
        right = lax.rem(me + 1, N_DEV)
        left = lax.rem(me + 3, N_DEV)

        barrier_sem = pltpu.get_barrier_semaphore()
        for nbr in (left, right):
            pl.semaphore_signal(
                barrier_sem, inc=1,
                device_id=(nbr,), device_id_type=pl.DeviceIdType.MESH,
            )
        pl.semaphore_wait(barrier_sem, 2)

        def hop(h):
            dq = pltpu.make_async_remote_copy(
                src_ref=wq_ref if h == 1 else comm_wq.at[h - 2],
                dst_ref=comm_wq.at[h - 1],
                send_sem=sq_send.at[h - 1],
                recv_sem=sq_recv.at[h - 1],
                device_id=(right,),
                device_id_type=pl.DeviceIdType.MESH,
            )
            do = pltpu.make_async_remote_copy(
                src_ref=wo_ref if h == 1 else comm_wo.at[h - 2],
                dst_ref=comm_wo.at[h - 1],
                send_sem=so_send.at[h - 1],
                recv_sem=so_recv.at[h - 1],
                device_id=(left,),
                device_id_type=pl.DeviceIdType.MESH,
            )
            dq.start()
            do.start()
            return dq, do

        def kv_dma(src, p, slot):
            g = lax.rem(me - p + N_DEV, N_DEV)
            d = pltpu.make_async_copy(
                src.at[me, :, pl.ds(g * GW, GW)],
                stage.at[slot],
                kv_sems.at[slot],
            )
            d.start()
            return d

        def wq_at(p):
            return wq_ref[:, :] if p == 0 else comm_wq[p - 1, :, :]

        def wo_at(q):
            return wo_ref[:, :] if q == 0 else comm_wo[q - 1, :, :]

        def attn(p):
            qbuf[:, :] = jnp.dot(
                x_ref[:, :], wq_at(p), preferred_element_type=jnp.float32
            ).astype(jnp.bfloat16)
            _attn_group(qbuf, kp, vp, ctx, p)

        def outproj(q, first=False):
            pblk = (N_DEV - q) % N_DEV
            contrib = jnp.dot(
                ctx[:, pblk * GW:(pblk + 1) * GW], wo_at(q),
                preferred_element_type=jnp.float32,
            )
            if first:
                out_ref[:, :] = contrib
            else:
                out_ref[:, :] = out_ref[:, :] + contrib

        dq1, do1 = hop(1)
        dk0 = kv_dma(k_hbm, 0, 0)
        dv0 = kv_dma(v_hbm, 0, 1)
        dk0.wait()
        _shuffle_block(stage, 0, kp, 0)
        dk1 = kv_dma(k_hbm, 1, 0)
        dv0.wait()
        _shuffle_block(stage, 1, vp, 0)
        dv1 = kv_dma(v_hbm, 1, 1)

        attn(0)
        outproj(0, first=True)

        dq1.wait_recv()
        do1.wait_recv()
        dq2, do2 = hop(2)
        dk1.wait()
        _shuffle_block(stage, 0, kp, 1)
        dk2 = kv_dma(k_hbm, 2, 0)
        dv1.wait()
        _shuffle_block(stage, 1, vp, 1)
        dv2 = kv_dma(v_hbm, 2, 1)

        attn(1)

        dq2.wait_recv()
        do2.wait_recv()
        dq3, do3 = hop(3)
        dk2.wait()
        _shuffle_block(stage, 0, kp, 2)
        dk3 = kv_dma(k_hbm, 3, 0)
        dv2.wait()
        _shuffle_block(stage, 1, vp, 2)
        dv3 = kv_dma(v_hbm, 3, 1)

        attn(2)
        outproj(2)

        dq3.wait_recv()
        do3.wait_recv()
        dk3.wait()
        _shuffle_block(stage, 0, kp, 3)
        dv3.wait()
        _shuffle_block(stage, 1, vp, 3)

        attn(3)
        outproj(1)
        outproj(3)

        for d in (dq1, do1, dq2, do2, dq3, do3):
            d.wait_send()

    out_p = pl.pallas_call(
        body,
        out_shape=jax.ShapeDtypeStruct((SQ, DM), jnp.float32),
        in_specs=[
            pl.BlockSpec(memory_space=pltpu.VMEM),
            pl.BlockSpec(memory_space=pltpu.VMEM),
            pl.BlockSpec(memory_space=pltpu.VMEM),
            pl.BlockSpec(memory_space=pltpu.MemorySpace.HBM),
            pl.BlockSpec(memory_space=pltpu.MemorySpace.HBM),
        ],
        out_specs=pl.BlockSpec(memory_space=pltpu.VMEM),
        scratch_shapes=[
            pltpu.VMEM((3, DM, GW), jnp.bfloat16),
            pltpu.VMEM((3, GW, DM), jnp.bfloat16),
            pltpu.VMEM((SQ, N_DEV * GW), jnp.bfloat16),
            pltpu.VMEM((SQ, N_DEV * GW), jnp.bfloat16),
            pltpu.VMEM((2, SQ, GW), jnp.float32),
            pltpu.VMEM((SQ, GW), jnp.bfloat16),
            pltpu.VMEM((SQ, N_DEV * GW), jnp.bfloat16),
            pltpu.SemaphoreType.DMA((3,)),
            pltpu.SemaphoreType.DMA((3,)),
            pltpu.SemaphoreType.DMA((3,)),
            pltpu.SemaphoreType.DMA((3,)),
            pltpu.SemaphoreType.DMA((2,)),
        ],
        compiler_params=pltpu.CompilerParams(
            collective_id=0,
            vmem_limit_bytes=62 * 1024 * 1024,
        ),
    )(xb, wq, wo, k2, v2)

    return _perm_rows(out_p)[None]


# device time: 177458 ns/iter; 1.7532x vs baseline; 1.3411x over previous
import jax
import jax.numpy as jnp
from jax import lax
from jax.experimental import pallas as pl
from jax.experimental.pallas import tpu as pltpu

import os
_ABLATE_ATTN = os.environ.get("ABLATE_ATTN") == "1"
_ABLATE_RING = os.environ.get("ABLATE_RING") == "1"
_ABLATE_KV = os.environ.get("ABLATE_KV") == "1"

N_DEV = 4
SQ = 1024
DM = 1024
GW = 1024
HG = 8
DH = 128
NR = 4
RB = SQ // NR
NB = 16
SCALE = 0.08838834764831843


def _perm_rows(a):
    c = a.shape[1]
    return a.reshape(4, 4, 64, c).swapaxes(0, 1).reshape(SQ, c)


def _attn_group(qbuf, kp, vp, ctx, p):
    col0 = p * GW
    for h in range(HG):
        qc = h * DH
        kc = col0 + h * DH
        for r in range(NR):
            row = r * RB
            if _ABLATE_ATTN:
                ctx[row:row + RB, kc:kc + DH] = (
                    qbuf[row:row + RB, qc:qc + DH]
                    + kp[row:row + RB, kc:kc + DH]
                    + vp[row:row + RB, kc:kc + DH])
                continue
            q = qbuf[row:row + RB, qc:qc + DH]
            k = kp[row:row + RB, kc:kc + DH]
            s = lax.dot_general(
                q, k, (((1,), (1,)), ((), ())),
                preferred_element_type=jnp.float32,
            )
            m = jnp.max(s, axis=1, keepdims=True)
            e = jnp.exp(s - m)
            w = (e / jnp.sum(e, axis=1, keepdims=True)).astype(jnp.bfloat16)
            v = vp[row:row + RB, kc:kc + DH]
            ctx[row:row + RB, kc:kc + DH] = jnp.dot(
                w, v, preferred_element_type=jnp.float32
            ).astype(jnp.bfloat16)


def _shuffle_block(stage, slot, dst, p):
    c0 = p * GW
    for b in range(NB):
        qi, r = b // 4, b % 4
        d0 = r * RB + qi * 64
        dst[d0:d0 + 64, c0:c0 + GW] = (
            stage[slot, b * 64:(b + 1) * 64, :].astype(jnp.bfloat16))


def kernel(x, Wq, K_ext, V_ext, Wo):
    xb = _perm_rows((x[0] * SCALE).astype(jnp.bfloat16))
    wq = Wq.astype(jnp.bfloat16)
    wo = Wo.astype(jnp.bfloat16)
    k2 = K_ext.reshape(N_DEV, SQ, N_DEV * GW)
    v2 = V_ext.reshape(N_DEV, SQ, N_DEV * GW)

    def body(x_ref, wq_ref, wo_ref, k_hbm, v_hbm, out_ref,
             comm_wq, comm_wo, kp, vp, stage, qbuf, ctx,
             sq_send, sq_recv, so_send, so_recv, kv_sems):
        me = lax.axis_index("i")
        right = lax.rem(me + 1, N_DEV)
        left = lax.rem(me + 3, N_DEV)

        barrier_sem = pltpu.get_barrier_semaphore()
        for nbr in (left, right):
            pl.semaphore_signal(
                barrier_sem, inc=1,
                device_id=(nbr,), device_id_type=pl.DeviceIdType.MESH,
            )
        pl.semaphore_wait(barrier_sem, 2)

        def hop(h):
            dq = pltpu.make_async_remote_copy(
                src_ref=wq_ref if h == 1 else comm_wq.at[h - 2],
                dst_ref=comm_wq.at[h - 1],
                send_sem=sq_send.at[h - 1],
                recv_sem=sq_recv.at[h - 1],
                device_id=(right,),
                device_id_type=pl.DeviceIdType.MESH,
            )
            do = pltpu.make_async_remote_copy(
                src_ref=wo_ref if h == 1 else comm_wo.at[h - 2],
                dst_ref=comm_wo.at[h - 1],
                send_sem=so_send.at[h - 1],
                recv_sem=so_recv.at[h - 1],
                device_id=(left,),
                device_id_type=pl.DeviceIdType.MESH,
            )
            dq.start()
            do.start()
            return dq, do

        def kv_dma(src, p, slot):
            g = lax.rem(me - p + N_DEV, N_DEV)
            d = pltpu.make_async_copy(
                src.at[me, :, pl.ds(g * GW, GW)],
                stage.at[slot],
                kv_sems.at[slot],
            )
            d.start()
            return d

        def wq_at(p):
            return wq_ref[:, :] if p == 0 else comm_wq[p - 1, :, :]

        def wo_at(q):
            return wo_ref[:, :] if q == 0 else comm_wo[q - 1, :, :]

        def attn(p):
            qbuf[:, :] = jnp.dot(
                x_ref[:, :], wq_at(p), preferred_element_type=jnp.float32
            ).astype(jnp.bfloat16)
            _attn_group(qbuf, kp, vp, ctx, p)

        def outproj(q, first=False):
            pblk = (N_DEV - q) % N_DEV
            contrib = jnp.dot(
                ctx[:, pblk * GW:(pblk + 1) * GW], wo_at(q),
                preferred_element_type=jnp.float32,
            )
            if first:
                out_ref[:, :] = contrib
            else:
                out_ref[:, :] = out_ref[:, :] + contrib

        if _ABLATE_RING:
            comm_wq[0, :, :] = wq_ref[:, :]
            comm_wq[1, :, :] = wq_ref[:, :]
            comm_wq[2, :, :] = wq_ref[:, :]
            comm_wo[0, :, :] = wo_ref[:, :]
            comm_wo[1, :, :] = wo_ref[:, :]
            comm_wo[2, :, :] = wo_ref[:, :]
            for p in range(N_DEV):
                if _ABLATE_KV:
                    kp[:, p * GW:(p + 1) * GW] = x_ref[:, :]
                    vp[:, p * GW:(p + 1) * GW] = x_ref[:, :]
                else:
                    d = kv_dma(k_hbm, p, 0)
                    d.wait()
                    _shuffle_block(stage, 0, kp, p)
                    d = kv_dma(v_hbm, p, 1)
                    d.wait()
                    _shuffle_block(stage, 1, vp, p)
                attn(p)
            outproj(0, first=True)
            outproj(1)
            outproj(2)
            outproj(3)
            return

        dq1, do1 = hop(1)
        dk0 = kv_dma(k_hbm, 0, 0)
        dv0 = kv_dma(v_hbm, 0, 1)
        dk0.wait()
        _shuffle_block(stage, 0, kp, 0)
        dk1 = kv_dma(k_hbm, 1, 0)
        dv0.wait()
        _shuffle_block(stage, 1, vp, 0)
        dv1 = kv_dma(v_hbm, 1, 1)

        attn(0)
        outproj(0, first=True)

        dq1.wait_recv()
        do1.wait_recv()
        dq2, do2 = hop(2)
        dk1.wait()
        _shuffle_block(stage, 0, kp, 1)
        dk2 = kv_dma(k_hbm, 2, 0)
        dv1.wait()
        _shuffle_block(stage, 1, vp, 1)
        dv2 = kv_dma(v_hbm, 2, 1)

        attn(1)

        dq2.wait_recv()
        do2.wait_recv()
        dq3, do3 = hop(3)
        dk2.wait()
        _shuffle_block(stage, 0, kp, 2)
        dk3 = kv_dma(k_hbm, 3, 0)
        dv2.wait()
        _shuffle_block(stage, 1, vp, 2)
        dv3 = kv_dma(v_hbm, 3, 1)

        attn(2)
        outproj(2)

        dq3.wait_recv()
        do3.wait_recv()
        dk3.wait()
        _shuffle_block(stage, 0, kp, 3)
        dv3.wait()
        _shuffle_block(stage, 1, vp, 3)

        attn(3)
        outproj(1)
        outproj(3)

        for d in (dq1, do1, dq2, do2, dq3, do3):
            d.wait_send()

    out_p = pl.pallas_call(
        body,
        out_shape=jax.ShapeDtypeStruct((SQ, DM), jnp.float32),
        in_specs=[
            pl.BlockSpec(memory_space=pltpu.VMEM),
            pl.BlockSpec(memory_space=pltpu.VMEM),
            pl.BlockSpec(memory_space=pltpu.VMEM),
            pl.BlockSpec(memory_space=pltpu.MemorySpace.HBM),
            pl.BlockSpec(memory_space=pltpu.MemorySpace.HBM),
        ],
        out_specs=pl.BlockSpec(memory_space=pltpu.VMEM),
        scratch_shapes=[
            pltpu.VMEM((3, DM, GW), jnp.bfloat16),
            pltpu.VMEM((3, GW, DM), jnp.bfloat16),
            pltpu.VMEM((SQ, N_DEV * GW), jnp.bfloat16),
            pltpu.VMEM((SQ, N_DEV * GW), jnp.bfloat16),
            pltpu.VMEM((2, SQ, GW), jnp.float32),
            pltpu.VMEM((SQ, GW), jnp.bfloat16),
            pltpu.VMEM((SQ, N_DEV * GW), jnp.bfloat16),
            pltpu.SemaphoreType.DMA((3,)),
            pltpu.SemaphoreType.DMA((3,)),
            pltpu.SemaphoreType.DMA((3,)),
            pltpu.SemaphoreType.DMA((3,)),
            pltpu.SemaphoreType.DMA((2,)),
        ],
        compiler_params=pltpu.CompilerParams(
            collective_id=0,
            vmem_limit_bytes=62 * 1024 * 1024,
        ),
    )(xb, wq, wo, k2, v2)

    return _perm_rows(out_p)[None]


# device time: 112908 ns/iter; 2.7555x vs baseline; 1.5717x over previous
import jax
import jax.numpy as jnp
from jax import lax
from jax.experimental import pallas as pl
from jax.experimental.pallas import tpu as pltpu

N_DEV = 4
SQ = 1024
DM = 1024
GW = 1024
HG = 8
DH = 128
NR = 4
RB = SQ // NR
NB = 16
SCALE = 0.08838834764831843


def _perm_rows(a):
    c = a.shape[1]
    return a.reshape(4, 4, 64, c).swapaxes(0, 1).reshape(SQ, c)


def kernel(x, Wq, K_ext, V_ext, Wo):
    xb = _perm_rows((x[0] * SCALE).astype(jnp.bfloat16))
    wq = Wq.astype(jnp.bfloat16)
    wo = Wo.astype(jnp.bfloat16)

    def body(x_ref, wq_ref, wo_ref, k_hbm, v_hbm, out_ref,
             comm_wq, comm_wo, kbuf, vbuf, qbuf, ctx,
             sq_send, sq_recv, so_send, so_recv, kv_sems):
        me = lax.axis_index("i")
        right = lax.rem(me + 1, N_DEV)
        left = lax.rem(me + 3, N_DEV)

        barrier_sem = pltpu.get_barrier_semaphore()
        for nbr in (left, right):
            pl.semaphore_signal(
                barrier_sem, inc=1,
                device_id=(nbr,), device_id_type=pl.DeviceIdType.MESH,
            )
        pl.semaphore_wait(barrier_sem, 2)

        def hop(h):
            dq = pltpu.make_async_remote_copy(
                src_ref=wq_ref if h == 1 else comm_wq.at[h - 2],
                dst_ref=comm_wq.at[h - 1],
                send_sem=sq_send.at[h - 1],
                recv_sem=sq_recv.at[h - 1],
                device_id=(right,),
                device_id_type=pl.DeviceIdType.MESH,
            )
            do = pltpu.make_async_remote_copy(
                src_ref=wo_ref if h == 1 else comm_wo.at[h - 2],
                dst_ref=comm_wo.at[h - 1],
                send_sem=so_send.at[h - 1],
                recv_sem=so_recv.at[h - 1],
                device_id=(left,),
                device_id_type=pl.DeviceIdType.MESH,
            )
            dq.start()
            do.start()
            return dq, do

        def kv_load(src, buf, p):
            g = lax.rem(me - p + N_DEV, N_DEV)
            slot = p % 2
            sem = kv_sems.at[0 if src is k_hbm else 1, slot]
            ds = []
            for b in range(NB):
                qi, r = b // 4, b % 4
                d0 = r * RB + qi * 64
                d = pltpu.make_async_copy(
                    src.at[me, pl.ds(b * 64, 64), pl.ds(g * HG, HG), :],
                    buf.at[slot, pl.ds(d0, 64), :, :],
                    sem,
                )
                d.start()
                ds.append(d)
            return ds

        def wq_at(p):
            return wq_ref[:, :] if p == 0 else comm_wq[p - 1, :, :]

        def wo_at(q):
            return wo_ref[:, :] if q == 0 else comm_wo[q - 1, :, :]

        def attn(p):
            slot = p % 2
            qbuf[:, :] = jnp.dot(
                x_ref[:, :], wq_at(p), preferred_element_type=jnp.float32
            ).astype(jnp.bfloat16)
            c0 = p * GW
            for h in range(HG):
                qc = h * DH
                kc = c0 + h * DH
                for r in range(NR):
                    row = r * RB
                    q = qbuf[row:row + RB, qc:qc + DH]
                    k = kbuf[slot, row:row + RB, h, :].astype(jnp.bfloat16)
                    s = lax.dot_general(
                        q, k, (((1,), (1,)), ((), ())),
                        preferred_element_type=jnp.float32,
                    )
                    m = jnp.max(s, axis=1, keepdims=True)
                    e = jnp.exp(s - m)
                    w = (e / jnp.sum(e, axis=1, keepdims=True)).astype(jnp.bfloat16)
                    v = vbuf[slot, row:row + RB, h, :].astype(jnp.bfloat16)
                    ctx[row:row + RB, kc:kc + DH] = jnp.dot(
                        w, v, preferred_element_type=jnp.float32
                    ).astype(jnp.bfloat16)

        def outproj(q, first=False):
            pblk = (N_DEV - q) % N_DEV
            contrib = jnp.dot(
                ctx[:, pblk * GW:(pblk + 1) * GW], wo_at(q),
                preferred_element_type=jnp.float32,
            )
            if first:
                out_ref[:, :] = contrib
            else:
                out_ref[:, :] = out_ref[:, :] + contrib

        dq1, do1 = hop(1)
        dk = kv_load(k_hbm, kbuf, 0)
        dv = kv_load(v_hbm, vbuf, 0)
        for d in dk + dv:
            d.wait()
        dk = kv_load(k_hbm, kbuf, 1)
        dv = kv_load(v_hbm, vbuf, 1)

        attn(0)
        outproj(0, first=True)

        dq1.wait_recv()
        do1.wait_recv()
        dq2, do2 = hop(2)
        for d in dk + dv:
            d.wait()
        dk = kv_load(k_hbm, kbuf, 2)
        dv = kv_load(v_hbm, vbuf, 2)

        attn(1)

        dq2.wait_recv()
        do2.wait_recv()
        dq3, do3 = hop(3)
        for d in dk + dv:
            d.wait()
        dk = kv_load(k_hbm, kbuf, 3)
        dv = kv_load(v_hbm, vbuf, 3)

        attn(2)
        outproj(2)

        dq3.wait_recv()
        do3.wait_recv()
        for d in dk + dv:
            d.wait()

        attn(3)
        outproj(1)
        outproj(3)

        for d in (dq1, do1, dq2, do2, dq3, do3):
            d.wait_send()

    out_p = pl.pallas_call(
        body,
        out_shape=jax.ShapeDtypeStruct((SQ, DM), jnp.float32),
        in_specs=[
            pl.BlockSpec(memory_space=pltpu.VMEM),
            pl.BlockSpec(memory_space=pltpu.VMEM),
            pl.BlockSpec(memory_space=pltpu.VMEM),
            pl.BlockSpec(memory_space=pltpu.MemorySpace.HBM),
            pl.BlockSpec(memory_space=pltpu.MemorySpace.HBM),
        ],
        out_specs=pl.BlockSpec(memory_space=pltpu.VMEM),
        scratch_shapes=[
            pltpu.VMEM((3, DM, GW), jnp.bfloat16),
            pltpu.VMEM((3, GW, DM), jnp.bfloat16),
            pltpu.VMEM((2, SQ, HG, DH), jnp.float32),
            pltpu.VMEM((2, SQ, HG, DH), jnp.float32),
            pltpu.VMEM((SQ, GW), jnp.bfloat16),
            pltpu.VMEM((SQ, N_DEV * GW), jnp.bfloat16),
            pltpu.SemaphoreType.DMA((3,)),
            pltpu.SemaphoreType.DMA((3,)),
            pltpu.SemaphoreType.DMA((3,)),
            pltpu.SemaphoreType.DMA((3,)),
            pltpu.SemaphoreType.DMA((2, 2)),
        ],
        compiler_params=pltpu.CompilerParams(
            collective_id=0,
            vmem_limit_bytes=62 * 1024 * 1024,
        ),
    )(xb, wq, wo, K_ext, V_ext)

    return _perm_rows(out_p)[None]
